# baseline (speedup 1.0000x reference)
import functools

import jax
import jax.numpy as jnp
from jax import lax
from jax.experimental import pallas as pl
from jax.experimental.pallas import tpu as pltpu
from jax.experimental.pallas import tpu_sc as plsc

_NC = 2
_NS = 16
_NW = _NC * _NS
_CH = 125


def _bmm_body(x_ref, w1_ref, w2_ref, b_ref, o_ref):
    blk2, d = x_ref.shape
    blk = blk2 // 2
    sb = 8
    b = b_ref[...]
    j = lax.broadcasted_iota(jnp.int32, (sb, sb * d), 1) // d
    nrow = lax.broadcasted_iota(jnp.int32, (sb, sb * d), 0)
    diag = j == nrow
    for half, w_ref in enumerate((w1_ref, w2_ref)):
        for g in range(blk // sb):
            r = half * blk + g * sb
            x = x_ref[pl.ds(r, sb), :]
            w = w_ref[pl.ds(g * sb, sb), :, :].reshape(sb * d, d)
            xe = jnp.where(diag, jnp.tile(x, (1, sb)), 0.0)
            o_ref[pl.ds(r, sb), :] = lax.dot_general(
                xe, w, (((1,), (0,)), ((), ())),
                preferred_element_type=jnp.float32) + b


def _node_transform(x, weight, bias):
    n, d = x.shape
    blk = 100
    return pl.pallas_call(
        _bmm_body,
        grid=(n // (2 * blk),),
        in_specs=[
            pl.BlockSpec((2 * blk, d), lambda i: (i, 0)),
            pl.BlockSpec((blk, d, d), lambda i: (2 * i, 0, 0)),
            pl.BlockSpec((blk, d, d), lambda i: (2 * i + 1, 0, 0)),
            pl.BlockSpec((1, d), lambda i: (0, 0)),
        ],
        out_specs=pl.BlockSpec((2 * blk, d), lambda i: (i, 0)),
        out_shape=jax.ShapeDtypeStruct((n, d), jnp.float32),
    )(x, weight, weight, bias.reshape(1, d))


def _make_agg(n, d, e):
    ch = _CH
    epw = e // _NW
    nch = epw // ch
    ni = 4
    ng = 2
    rps = (n // _NS) // 8 * 8
    rem = n - _NS * rps
    mesh = plsc.VectorSubcoreMesh(core_axis_name="c", subcore_axis_name="s")

    @functools.partial(
        pl.kernel,
        mesh=mesh,
        out_type=jax.ShapeDtypeStruct((_NC, n, d), jnp.float32),
        scratch_types=[
            pltpu.VMEM((ni, ch), jnp.int32),
            pltpu.VMEM((ni, ch), jnp.int32),
            pltpu.VMEM((ng, ch, d), jnp.float32),
            pltpu.VMEM_SHARED((n, d), jnp.float32),
        ]
        + [pltpu.SemaphoreType.DMA for _ in range(ni)]
        + [pltpu.SemaphoreType.DMA for _ in range(ng)],
    )
    def agg(out_hbm, row_hbm, col_hbm, parts_hbm, rowb, colb, rbuf, acc,
            *sems):
        isem = sems[:ni]
        gsem = sems[ni:]
        c = lax.axis_index("c")
        s = lax.axis_index("s")
        wid = c * _NS + s

        def load_idx(i, slot):
            pltpu.async_copy(row_hbm.at[wid, i], rowb.at[slot], isem[slot])
            pltpu.async_copy(col_hbm.at[wid, i], colb.at[slot], isem[slot])

        def wait_idx(slot):
            pltpu.make_async_copy(row_hbm.at[wid, 0], rowb.at[slot],
                                  isem[slot]).wait()
            pltpu.make_async_copy(col_hbm.at[wid, 0], colb.at[slot],
                                  isem[slot]).wait()

        def start_gather(islot, gslot):
            pltpu.async_copy(out_hbm.at[rowb.at[islot]], rbuf.at[gslot],
                             gsem[gslot])

        def wait_gather(gslot):
            pltpu.make_async_copy(out_hbm.at[rowb.at[0]], rbuf.at[gslot],
                                  gsem[gslot]).wait()

        for b in range(ni):
            load_idx(b, b)
        pltpu.sync_copy(out_hbm.at[pl.ds(s * rps, rps)],
                        acc.at[pl.ds(s * rps, rps)])
        if rem:
            @pl.when(s == 0)
            def _init_tail():
                pltpu.sync_copy(out_hbm.at[pl.ds(_NS * rps, rem)],
                                acc.at[pl.ds(_NS * rps, rem)])
        plsc.subcore_barrier()

        for b in range(ng):
            wait_idx(b)
            start_gather(b, b)

        def outer(t, carry):
            for b in range(ni):
                i = t * ni + b
                r2 = b % ng
                r4 = b
                wait_gather(r2)
                pltpu.sync_copy(rbuf.at[r2], acc.at[colb.at[r4]], add=True)

                @pl.when(i + ni < nch)
                def _next_idx():
                    load_idx(i + ni, r4)

                @pl.when(i + ng < nch)
                def _next_gather():
                    wait_idx((r4 + ng) % ni)
                    start_gather((r4 + ng) % ni, r2)
            return carry

        lax.fori_loop(0, nch // ni, outer, 0)
        plsc.subcore_barrier()
        pltpu.sync_copy(acc.at[pl.ds(s * rps, rps)],
                        parts_hbm.at[c, pl.ds(s * rps, rps)])
        if rem:
            @pl.when(s == 0)
            def _out_tail():
                pltpu.sync_copy(acc.at[pl.ds(_NS * rps, rem)],
                                parts_hbm.at[c, pl.ds(_NS * rps, rem)])

    return agg


def _combine_body(p_ref, o_ref, out_ref):
    out_ref[...] = p_ref[0] + p_ref[1] - o_ref[...]


def _combine(parts, out):
    n, d = out.shape
    blk = 1000
    return pl.pallas_call(
        _combine_body,
        grid=(n // blk,),
        in_specs=[
            pl.BlockSpec((2, blk, d), lambda i: (0, i, 0)),
            pl.BlockSpec((blk, d), lambda i: (i, 0)),
        ],
        out_specs=pl.BlockSpec((blk, d), lambda i: (i, 0)),
        out_shape=jax.ShapeDtypeStruct((n, d), jnp.float32),
    )(parts, out)


def kernel(x, edge_index, weight, bias):
    n, d = x.shape
    e = edge_index.shape[1]
    out = _node_transform(x, weight, bias)
    epw = e // _NW
    row = edge_index[0].astype(jnp.int32).reshape(_NW, epw // _CH, _CH)
    col = edge_index[1].astype(jnp.int32).reshape(_NW, epw // _CH, _CH)
    parts = _make_agg(n, d, e)(out, row, col)
    return _combine(parts, out)


# speedup vs baseline: 1.0079x; 1.0079x over previous
import functools

import jax
import jax.numpy as jnp
from jax import lax
from jax.experimental import pallas as pl
from jax.experimental.pallas import tpu as pltpu
from jax.experimental.pallas import tpu_sc as plsc

_NC = 2
_NS = 16
_NW = _NC * _NS
_CH = 125


def _bmm_body(x_ref, w_ref, b_ref, o_ref):
    blk, d = x_ref.shape
    sb = 8
    b = b_ref[...]
    j = lax.broadcasted_iota(jnp.int32, (sb, sb * d), 1) // d
    nrow = lax.broadcasted_iota(jnp.int32, (sb, sb * d), 0)
    diag = j == nrow
    for g in range(blk // sb):
        x = x_ref[pl.ds(g * sb, sb), :]
        w = w_ref[pl.ds(g * sb, sb), :, :].reshape(sb * d, d)
        xe = jnp.where(diag, jnp.tile(x, (1, sb)), 0.0)
        o_ref[pl.ds(g * sb, sb), :] = lax.dot_general(
            xe, w, (((1,), (0,)), ((), ())),
            preferred_element_type=jnp.float32) + b


def _node_transform(x, weight, bias):
    n, d = x.shape
    blk = 200
    return pl.pallas_call(
        _bmm_body,
        grid=(n // blk,),
        in_specs=[
            pl.BlockSpec((blk, d), lambda i: (i, 0)),
            pl.BlockSpec((blk, d, d), lambda i: (i, 0, 0)),
            pl.BlockSpec((1, d), lambda i: (0, 0)),
        ],
        out_specs=pl.BlockSpec((blk, d), lambda i: (i, 0)),
        out_shape=jax.ShapeDtypeStruct((n, d), jnp.float32),
    )(x, weight, bias.reshape(1, d))


def _make_agg(n, d, e):
    ch = _CH
    epw = e // _NW
    nch = epw // ch
    ni = 8
    ng = 2
    rps = (n // _NS) // 8 * 8
    rem = n - _NS * rps
    mesh = plsc.VectorSubcoreMesh(core_axis_name="c", subcore_axis_name="s")

    @functools.partial(
        pl.kernel,
        mesh=mesh,
        out_type=jax.ShapeDtypeStruct((_NC, n, d), jnp.float32),
        scratch_types=[
            pltpu.VMEM((ni, ch), jnp.int32),
            pltpu.VMEM((ni, ch), jnp.int32),
            pltpu.VMEM((ng, ch, d), jnp.float32),
            pltpu.VMEM_SHARED((n, d), jnp.float32),
        ]
        + [pltpu.SemaphoreType.DMA for _ in range(ni)]
        + [pltpu.SemaphoreType.DMA for _ in range(ng)],
    )
    def agg(out_hbm, row_hbm, col_hbm, parts_hbm, rowb, colb, rbuf, acc,
            *sems):
        isem = sems[:ni]
        gsem = sems[ni:]
        c = lax.axis_index("c")
        s = lax.axis_index("s")
        wid = c * _NS + s

        def load_idx(i, slot):
            pltpu.async_copy(row_hbm.at[wid, i], rowb.at[slot], isem[slot])
            pltpu.async_copy(col_hbm.at[wid, i], colb.at[slot], isem[slot])

        def wait_idx(slot):
            pltpu.make_async_copy(row_hbm.at[wid, 0], rowb.at[slot],
                                  isem[slot]).wait()
            pltpu.make_async_copy(col_hbm.at[wid, 0], colb.at[slot],
                                  isem[slot]).wait()

        def start_gather(islot, gslot):
            pltpu.async_copy(out_hbm.at[rowb.at[islot]], rbuf.at[gslot],
                             gsem[gslot])

        def wait_gather(gslot):
            pltpu.make_async_copy(out_hbm.at[rowb.at[0]], rbuf.at[gslot],
                                  gsem[gslot]).wait()

        for b in range(ni):
            load_idx(b, b)
        pltpu.sync_copy(out_hbm.at[pl.ds(s * rps, rps)],
                        acc.at[pl.ds(s * rps, rps)])
        if rem:
            @pl.when(s == 0)
            def _init_tail():
                pltpu.sync_copy(out_hbm.at[pl.ds(_NS * rps, rem)],
                                acc.at[pl.ds(_NS * rps, rem)])
        plsc.subcore_barrier()

        for b in range(ng):
            wait_idx(b)
            start_gather(b, b)

        def outer(t, carry):
            for b in range(ni):
                i = t * ni + b
                r2 = b % ng
                r4 = b
                wait_gather(r2)
                pltpu.sync_copy(rbuf.at[r2], acc.at[colb.at[r4]], add=True)

                @pl.when(i + ni < nch)
                def _next_idx():
                    load_idx(i + ni, r4)

                @pl.when(i + ng < nch)
                def _next_gather():
                    wait_idx((r4 + ng) % ni)
                    start_gather((r4 + ng) % ni, r2)
            return carry

        lax.fori_loop(0, nch // ni, outer, 0)
        plsc.subcore_barrier()
        pltpu.sync_copy(acc.at[pl.ds(s * rps, rps)],
                        parts_hbm.at[c, pl.ds(s * rps, rps)])
        if rem:
            @pl.when(s == 0)
            def _out_tail():
                pltpu.sync_copy(acc.at[pl.ds(_NS * rps, rem)],
                                parts_hbm.at[c, pl.ds(_NS * rps, rem)])

    return agg


def _combine_body(p_ref, o_ref, out_ref):
    out_ref[...] = p_ref[0] + p_ref[1] - o_ref[...]


def _combine(parts, out):
    n, d = out.shape
    blk = 2000
    return pl.pallas_call(
        _combine_body,
        grid=(n // blk,),
        in_specs=[
            pl.BlockSpec((2, blk, d), lambda i: (0, i, 0)),
            pl.BlockSpec((blk, d), lambda i: (i, 0)),
        ],
        out_specs=pl.BlockSpec((blk, d), lambda i: (i, 0)),
        out_shape=jax.ShapeDtypeStruct((n, d), jnp.float32),
    )(parts, out)


def kernel(x, edge_index, weight, bias):
    n, d = x.shape
    e = edge_index.shape[1]
    out = _node_transform(x, weight, bias)
    epw = e // _NW
    row = edge_index[0].astype(jnp.int32).reshape(_NW, epw // _CH, _CH)
    col = edge_index[1].astype(jnp.int32).reshape(_NW, epw // _CH, _CH)
    parts = _make_agg(n, d, e)(out, row, col)
    return _combine(parts, out)


# speedup vs baseline: 1.0256x; 1.0176x over previous
import functools

import jax
import jax.numpy as jnp
from jax import lax
from jax.experimental import pallas as pl
from jax.experimental.pallas import tpu as pltpu
from jax.experimental.pallas import tpu_sc as plsc

_NC = 2
_NS = 16
_NW = _NC * _NS
_CH = 50


def _bmm_body(x_ref, w_ref, b_ref, o_ref):
    blk, d = x_ref.shape
    sb = 8
    b = b_ref[...]
    j = lax.broadcasted_iota(jnp.int32, (sb, sb * d), 1) // d
    nrow = lax.broadcasted_iota(jnp.int32, (sb, sb * d), 0)
    diag = j == nrow
    for g in range(blk // sb):
        x = x_ref[pl.ds(g * sb, sb), :]
        w = w_ref[pl.ds(g * sb, sb), :, :].reshape(sb * d, d)
        xe = jnp.where(diag, jnp.tile(x, (1, sb)), 0.0)
        o_ref[pl.ds(g * sb, sb), :] = lax.dot_general(
            xe, w, (((1,), (0,)), ((), ())),
            preferred_element_type=jnp.float32) + b


def _node_transform(x, weight, bias):
    n, d = x.shape
    blk = 200
    return pl.pallas_call(
        _bmm_body,
        grid=(n // blk,),
        in_specs=[
            pl.BlockSpec((blk, d), lambda i: (i, 0)),
            pl.BlockSpec((blk, d, d), lambda i: (i, 0, 0)),
            pl.BlockSpec((1, d), lambda i: (0, 0)),
        ],
        out_specs=pl.BlockSpec((blk, d), lambda i: (i, 0)),
        out_shape=jax.ShapeDtypeStruct((n, d), jnp.float32),
    )(x, weight, bias.reshape(1, d))


def _make_agg(n, d, e):
    ch = _CH
    epw = e // _NW
    nch = epw // ch
    ni = 8
    ng = 4
    rps = (n // _NS) // 8 * 8
    rem = n - _NS * rps
    mesh = plsc.VectorSubcoreMesh(core_axis_name="c", subcore_axis_name="s")

    @functools.partial(
        pl.kernel,
        mesh=mesh,
        out_type=jax.ShapeDtypeStruct((_NC, n, d), jnp.float32),
        scratch_types=[
            pltpu.VMEM((ni, ch), jnp.int32),
            pltpu.VMEM((ni, ch), jnp.int32),
            pltpu.VMEM((ng, ch, d), jnp.float32),
            pltpu.VMEM_SHARED((n, d), jnp.float32),
        ]
        + [pltpu.SemaphoreType.DMA for _ in range(ni)]
        + [pltpu.SemaphoreType.DMA for _ in range(ng)],
    )
    def agg(out_hbm, row_hbm, col_hbm, parts_hbm, rowb, colb, rbuf, acc,
            *sems):
        isem = sems[:ni]
        gsem = sems[ni:]
        c = lax.axis_index("c")
        s = lax.axis_index("s")
        wid = c * _NS + s

        def load_idx(i, slot):
            pltpu.async_copy(row_hbm.at[wid, i], rowb.at[slot], isem[slot])
            pltpu.async_copy(col_hbm.at[wid, i], colb.at[slot], isem[slot])

        def wait_idx(slot):
            pltpu.make_async_copy(row_hbm.at[wid, 0], rowb.at[slot],
                                  isem[slot]).wait()
            pltpu.make_async_copy(col_hbm.at[wid, 0], colb.at[slot],
                                  isem[slot]).wait()

        def start_gather(islot, gslot):
            pltpu.async_copy(out_hbm.at[rowb.at[islot]], rbuf.at[gslot],
                             gsem[gslot])

        def wait_gather(gslot):
            pltpu.make_async_copy(out_hbm.at[rowb.at[0]], rbuf.at[gslot],
                                  gsem[gslot]).wait()

        for b in range(ni):
            load_idx(b, b)
        pltpu.sync_copy(out_hbm.at[pl.ds(s * rps, rps)],
                        acc.at[pl.ds(s * rps, rps)])
        if rem:
            @pl.when(s == 0)
            def _init_tail():
                pltpu.sync_copy(out_hbm.at[pl.ds(_NS * rps, rem)],
                                acc.at[pl.ds(_NS * rps, rem)])
        plsc.subcore_barrier()

        for b in range(ng):
            wait_idx(b)
            start_gather(b, b)

        def outer(t, carry):
            for b in range(ni):
                i = t * ni + b
                r2 = b % ng
                r4 = b
                wait_gather(r2)
                pltpu.sync_copy(rbuf.at[r2], acc.at[colb.at[r4]], add=True)

                @pl.when(i + ni < nch)
                def _next_idx():
                    load_idx(i + ni, r4)

                @pl.when(i + ng < nch)
                def _next_gather():
                    wait_idx((r4 + ng) % ni)
                    start_gather((r4 + ng) % ni, r2)
            return carry

        lax.fori_loop(0, nch // ni, outer, 0)
        plsc.subcore_barrier()
        pltpu.sync_copy(acc.at[pl.ds(s * rps, rps)],
                        parts_hbm.at[c, pl.ds(s * rps, rps)])
        if rem:
            @pl.when(s == 0)
            def _out_tail():
                pltpu.sync_copy(acc.at[pl.ds(_NS * rps, rem)],
                                parts_hbm.at[c, pl.ds(_NS * rps, rem)])

    return agg


def _combine_body(p_ref, o_ref, out_ref):
    out_ref[...] = p_ref[0] + p_ref[1] - o_ref[...]


def _combine(parts, out):
    n, d = out.shape
    blk = 2000
    return pl.pallas_call(
        _combine_body,
        grid=(n // blk,),
        in_specs=[
            pl.BlockSpec((2, blk, d), lambda i: (0, i, 0)),
            pl.BlockSpec((blk, d), lambda i: (i, 0)),
        ],
        out_specs=pl.BlockSpec((blk, d), lambda i: (i, 0)),
        out_shape=jax.ShapeDtypeStruct((n, d), jnp.float32),
    )(parts, out)


def kernel(x, edge_index, weight, bias):
    n, d = x.shape
    e = edge_index.shape[1]
    out = _node_transform(x, weight, bias)
    epw = e // _NW
    row = edge_index[0].astype(jnp.int32).reshape(_NW, epw // _CH, _CH)
    col = edge_index[1].astype(jnp.int32).reshape(_NW, epw // _CH, _CH)
    parts = _make_agg(n, d, e)(out, row, col)
    return _combine(parts, out)
